# baseline (speedup 1.0000x reference)
import jax
import jax.numpy as jnp
from jax import lax
from jax.experimental import pallas as pl
from jax.experimental.pallas import tpu as pltpu
from jax.experimental.pallas import tpu_sc as plsc

NUM_EMB = 1000000
DIM = 32
B1 = 16384
S = 50
B_TOTAL = B1 * S
NC, NS = 2, 16
NW = NC * NS
IDX_PER_GROUP = 128
GROUPS_PER_W = B_TOTAL // (NW * IDX_PER_GROUP)
G_PER_CHUNK = 10
CHUNK_ROWS = G_PER_CHUNK * IDX_PER_GROUP
N_CHUNKS = GROUPS_PER_W // G_PER_CHUNK
ROWS_PER_W = GROUPS_PER_W * IDX_PER_GROUP


def _gather_body(idx_hbm, table_hbm, out_hbm, idx_v, buf0, buf1, gsem0, gsem1):
    c = lax.axis_index("c")
    s = lax.axis_index("s")
    wid = s * NC + c
    gbase = wid * GROUPS_PER_W
    rbase = wid * ROWS_PER_W

    pltpu.sync_copy(idx_hbm.at[pl.ds(gbase, GROUPS_PER_W)], idx_v)

    bufs = (buf0, buf1)
    gsems = (gsem0, gsem1)

    def fire(chunk, b):
        for j in range(G_PER_CHUNK):
            pltpu.async_copy(
                table_hbm.at[idx_v.at[chunk * G_PER_CHUNK + j]],
                bufs[b].at[pl.ds(j * IDX_PER_GROUP, IDX_PER_GROUP)],
                gsems[b],
            )

    def drain(b):
        pltpu.make_async_copy(
            out_hbm.at[pl.ds(0, CHUNK_ROWS)], bufs[b], gsems[b]
        ).wait()

    fire(0, 0)
    fire(1, 1)

    def step(it, carry):
        chunk = it * 2
        for b in range(2):
            cc = chunk + b
            drain(b)
            pltpu.sync_copy(
                bufs[b], out_hbm.at[pl.ds(rbase + cc * CHUNK_ROWS, CHUNK_ROWS)]
            )

            @pl.when(cc + 2 < N_CHUNKS)
            def _():
                fire(cc + 2, b)

        return carry

    lax.fori_loop(0, N_CHUNKS // 2, step, 0)


def _tr_body(x_ref, o_ref):
    x = x_ref[...]
    tok = x.reshape(IDX_PER_GROUP, DIM)
    o_ref[...] = jnp.transpose(tok).reshape(1, DIM, IDX_PER_GROUP)


@jax.jit
def _emb_lookup(idx_flat, weight):
    mesh = plsc.VectorSubcoreMesh(
        core_axis_name="c", subcore_axis_name="s", num_cores=NC, num_subcores=NS
    )
    gather = pl.kernel(
        _gather_body,
        out_type=jax.ShapeDtypeStruct((B_TOTAL, DIM), jnp.float32),
        mesh=mesh,
        scratch_types=[
            pltpu.VMEM((GROUPS_PER_W, IDX_PER_GROUP), jnp.int32),
            pltpu.VMEM((CHUNK_ROWS, DIM), jnp.float32),
            pltpu.VMEM((CHUNK_ROWS, DIM), jnp.float32),
            pltpu.SemaphoreType.DMA,
            pltpu.SemaphoreType.DMA,
        ],
        compiler_params=pltpu.CompilerParams(use_tc_tiling_on_sc=False),
    )
    rows = gather(idx_flat, weight)

    rows_v = rows.reshape(B_TOTAL * DIM // 128, 128)

    n_b1_blk = B1 // IDX_PER_GROUP
    out_t = pl.pallas_call(
        _tr_body,
        grid=(S, n_b1_blk),
        in_specs=[
            pl.BlockSpec((32, 128), lambda si, bi: (si * n_b1_blk + bi, 0))
        ],
        out_specs=pl.BlockSpec((1, DIM, IDX_PER_GROUP), lambda si, bi: (si, 0, bi)),
        out_shape=jax.ShapeDtypeStruct((S, DIM, B1), jnp.float32),
    )(rows_v)

    return jnp.transpose(out_t, (2, 0, 1))


def kernel(input_, weight):
    idx_flat = (
        input_.astype(jnp.int32).T.reshape(B_TOTAL // IDX_PER_GROUP, IDX_PER_GROUP)
    )
    return _emb_lookup(idx_flat, weight)


# speedup vs baseline: 2.4596x; 2.4596x over previous
import jax
import jax.numpy as jnp
from jax import lax
from jax.experimental import pallas as pl
from jax.experimental.pallas import tpu as pltpu
from jax.experimental.pallas import tpu_sc as plsc

NUM_EMB = 1000000
DIM = 32
B1 = 16384
S = 50
B_TOTAL = B1 * S
NC, NS = 2, 16
NW = NC * NS
IDX_PER_GROUP = 128
GROUPS_PER_W = B_TOTAL // (NW * IDX_PER_GROUP)
G_PER_CHUNK = 10
CHUNK_ROWS = G_PER_CHUNK * IDX_PER_GROUP
N_CHUNKS = GROUPS_PER_W // G_PER_CHUNK
ROWS_PER_W = GROUPS_PER_W * IDX_PER_GROUP


def _gather_body(idx_hbm, table_hbm, out_hbm, idx_v, buf0, buf1, gsem0, gsem1):
    c = lax.axis_index("c")
    s = lax.axis_index("s")
    wid = s * NC + c
    gbase = wid * GROUPS_PER_W
    rbase = wid * ROWS_PER_W

    pltpu.sync_copy(idx_hbm.at[pl.ds(gbase, GROUPS_PER_W)], idx_v)

    bufs = (buf0, buf1)
    gsems = (gsem0, gsem1)

    def fire(chunk, b):
        for j in range(G_PER_CHUNK):
            pltpu.async_copy(
                table_hbm.at[idx_v.at[chunk * G_PER_CHUNK + j]],
                bufs[b].at[pl.ds(j * IDX_PER_GROUP, IDX_PER_GROUP)],
                gsems[b],
            )

    def drain(b):
        pltpu.make_async_copy(
            out_hbm.at[pl.ds(0, CHUNK_ROWS)], bufs[b], gsems[b]
        ).wait()

    fire(0, 0)
    fire(1, 1)

    def step(it, carry):
        chunk = it * 2
        for b in range(2):
            cc = chunk + b
            drain(b)
            pltpu.sync_copy(
                bufs[b], out_hbm.at[pl.ds(rbase + cc * CHUNK_ROWS, CHUNK_ROWS)]
            )

            @pl.when(cc + 2 < N_CHUNKS)
            def _():
                fire(cc + 2, b)

        return carry

    lax.fori_loop(0, N_CHUNKS // 2, step, 0)


RB = 128
TB = RB * 4
NBB = B1 // TB


def _tr_body(s_ref, x_ref, o_ref):
    x = x_ref[...]
    acc = jnp.zeros((DIM, TB), jnp.float32)
    for q in range(4):
        xq = x[:, 32 * q:32 * q + 32]
        acc = acc + lax.dot_general(
            xq, s_ref[q],
            dimension_numbers=(((0,), (0,)), ((), ())),
            preferred_element_type=jnp.float32,
        )
    o_ref[...] = acc.reshape(1, DIM, TB)


def _make_smats():
    i = jnp.arange(RB)
    s = jnp.zeros((4, RB, TB), jnp.float32)
    for q in range(4):
        s = s.at[q, i, 4 * i + q].set(1.0)
    return s


@jax.jit
def _emb_lookup(idx_flat, weight):
    mesh = plsc.VectorSubcoreMesh(
        core_axis_name="c", subcore_axis_name="s", num_cores=NC, num_subcores=NS
    )
    gather = pl.kernel(
        _gather_body,
        out_type=jax.ShapeDtypeStruct((B_TOTAL, DIM), jnp.float32),
        mesh=mesh,
        scratch_types=[
            pltpu.VMEM((GROUPS_PER_W, IDX_PER_GROUP), jnp.int32),
            pltpu.VMEM((CHUNK_ROWS, DIM), jnp.float32),
            pltpu.VMEM((CHUNK_ROWS, DIM), jnp.float32),
            pltpu.SemaphoreType.DMA,
            pltpu.SemaphoreType.DMA,
        ],
        compiler_params=pltpu.CompilerParams(use_tc_tiling_on_sc=False),
    )
    rows = gather(idx_flat, weight)

    rows_v = rows.reshape(B_TOTAL * DIM // 128, 128)

    out_t = pl.pallas_call(
        _tr_body,
        grid=(S, NBB),
        in_specs=[
            pl.BlockSpec((4, RB, TB), lambda si, bi: (0, 0, 0)),
            pl.BlockSpec((RB, 128), lambda si, bi: (si * NBB + bi, 0)),
        ],
        out_specs=pl.BlockSpec((1, DIM, TB), lambda si, bi: (si, 0, bi)),
        out_shape=jax.ShapeDtypeStruct((S, DIM, B1), jnp.float32),
    )(_make_smats(), rows_v)

    return jnp.transpose(out_t, (2, 0, 1))


def kernel(input_, weight):
    idx_flat = (
        input_.astype(jnp.int32).T.reshape(B_TOTAL // IDX_PER_GROUP, IDX_PER_GROUP)
    )
    return _emb_lookup(idx_flat, weight)


# speedup vs baseline: 4.2748x; 1.7380x over previous
import jax
import jax.numpy as jnp
from jax import lax
from jax.experimental import pallas as pl
from jax.experimental.pallas import tpu as pltpu
from jax.experimental.pallas import tpu_sc as plsc

NUM_EMB = 1000000
DIM = 32
B1 = 16384
S = 50
B_TOTAL = B1 * S
NC, NS = 2, 16
NW = NC * NS
IDX_PER_GROUP = 128
GROUPS_PER_W = B_TOTAL // (NW * IDX_PER_GROUP)
G_PER_CHUNK = 8
CHUNK_ROWS = G_PER_CHUNK * IDX_PER_GROUP
N_CHUNKS = GROUPS_PER_W // G_PER_CHUNK
N_BLOCKS = B_TOTAL // 512


def _gather_body(idx_hbm, table_hbm, out_hbm, idx_v, buf0, buf1, gsem0, gsem1):
    c = lax.axis_index("c")
    s = lax.axis_index("s")
    wid = s * NC + c
    gbase = wid * GROUPS_PER_W
    bbase = wid * (GROUPS_PER_W // 4)

    pltpu.sync_copy(idx_hbm.at[pl.ds(gbase, GROUPS_PER_W)], idx_v)

    bufs = (buf0, buf1)
    gsems = (gsem0, gsem1)

    def fire(chunk, b):
        for j in range(G_PER_CHUNK):
            pltpu.async_copy(
                table_hbm.at[idx_v.at[chunk * G_PER_CHUNK + j]],
                bufs[b].at[pl.ds(j * IDX_PER_GROUP, IDX_PER_GROUP)],
                gsems[b],
            )

    def drain(b):
        pltpu.make_async_copy(
            table_hbm.at[pl.ds(0, CHUNK_ROWS)], bufs[b], gsems[b]
        ).wait()

    def scatter(chunk, b):
        for j in range(G_PER_CHUNK):
            blk = bbase + chunk * (G_PER_CHUNK // 4) + (j // 4)
            pltpu.sync_copy(
                bufs[b].at[pl.ds(j * IDX_PER_GROUP, IDX_PER_GROUP)],
                out_hbm.at[blk, :, j % 4, :],
            )

    fire(0, 0)
    fire(1, 1)

    def step(it, carry):
        chunk = it * 2
        for b in range(2):
            cc = chunk + b
            drain(b)
            scatter(cc, b)

            @pl.when(cc + 2 < N_CHUNKS)
            def _():
                fire(cc + 2, b)

        return carry

    lax.fori_loop(0, N_CHUNKS // 2, step, 0)
    drain(0)
    scatter(N_CHUNKS - 1, 0)


RB = 512
NBB = B1 // (4 * RB)


def _tr_body(x_ref, o_ref):
    x = x_ref[...]
    for q in range(4):
        t = jnp.transpose(x[:, 32 * q:32 * q + 32])
        for blk in range(4):
            o_ref[0, :, blk * 512 + q * 128:blk * 512 + (q + 1) * 128] = (
                t[:, blk * 128:(blk + 1) * 128]
            )


@jax.jit
def _emb_lookup(idx_flat, weight):
    mesh = plsc.VectorSubcoreMesh(
        core_axis_name="c", subcore_axis_name="s", num_cores=NC, num_subcores=NS
    )
    gather = pl.kernel(
        _gather_body,
        out_type=jax.ShapeDtypeStruct((N_BLOCKS, IDX_PER_GROUP, 4, DIM), jnp.float32),
        mesh=mesh,
        scratch_types=[
            pltpu.VMEM((GROUPS_PER_W, IDX_PER_GROUP), jnp.int32),
            pltpu.VMEM((CHUNK_ROWS, DIM), jnp.float32),
            pltpu.VMEM((CHUNK_ROWS, DIM), jnp.float32),
            pltpu.SemaphoreType.DMA,
            pltpu.SemaphoreType.DMA,
        ],
        compiler_params=pltpu.CompilerParams(use_tc_tiling_on_sc=False),
    )
    rows = gather(idx_flat, weight)

    rows_v = rows.reshape(B_TOTAL * DIM // 128, 128)

    out_t = pl.pallas_call(
        _tr_body,
        grid=(S, NBB),
        in_specs=[
            pl.BlockSpec((RB, 128), lambda si, bi: (si * NBB + bi, 0)),
        ],
        out_specs=pl.BlockSpec((1, DIM, 4 * RB), lambda si, bi: (si, 0, bi)),
        out_shape=jax.ShapeDtypeStruct((S, DIM, B1), jnp.float32),
    )(rows_v)

    return jnp.transpose(out_t, (2, 0, 1))


def kernel(input_, weight):
    idx_flat = (
        input_.astype(jnp.int32).T.reshape(B_TOTAL // IDX_PER_GROUP, IDX_PER_GROUP)
    )
    return _emb_lookup(idx_flat, weight)


# speedup vs baseline: 4.6255x; 1.0820x over previous
import jax
import jax.numpy as jnp
from jax import lax
from jax.experimental import pallas as pl
from jax.experimental.pallas import tpu as pltpu
from jax.experimental.pallas import tpu_sc as plsc

NUM_EMB = 1000000
DIM = 32
B1 = 16384
S = 50
B_TOTAL = B1 * S
NC, NS = 2, 16
NW = NC * NS
IDX_PER_GROUP = 128
GROUPS_PER_W = B_TOTAL // (NW * IDX_PER_GROUP)
G_PER_CHUNK = 8
CHUNK_ROWS = G_PER_CHUNK * IDX_PER_GROUP
N_CHUNKS = GROUPS_PER_W // G_PER_CHUNK
N_BLOCKS = B_TOTAL // 512


def _gather_body(idx_hbm, table_hbm, out_hbm, idx_v, buf0, buf1, gsem0, gsem1):
    c = lax.axis_index("c")
    s = lax.axis_index("s")
    wid = s * NC + c
    gbase = wid * GROUPS_PER_W
    bbase = wid * (GROUPS_PER_W // 4)

    pltpu.sync_copy(idx_hbm.at[pl.ds(gbase, GROUPS_PER_W)], idx_v)

    bufs = (buf0, buf1)
    gsems = (gsem0, gsem1)

    def fire(chunk, b):
        for j in range(G_PER_CHUNK):
            pltpu.async_copy(
                table_hbm.at[idx_v.at[chunk * G_PER_CHUNK + j]],
                bufs[b].at[pl.ds(j * IDX_PER_GROUP, IDX_PER_GROUP)],
                gsems[b],
            )

    def drain(b):
        pltpu.make_async_copy(
            table_hbm.at[pl.ds(0, CHUNK_ROWS)], bufs[b], gsems[b]
        ).wait()

    def scatter(chunk, b):
        for j in range(G_PER_CHUNK):
            blk = bbase + chunk * (G_PER_CHUNK // 4) + (j // 4)
            pltpu.sync_copy(
                bufs[b].at[pl.ds(j * IDX_PER_GROUP, IDX_PER_GROUP)],
                out_hbm.at[blk, :, j % 4, :],
            )

    fire(0, 0)
    fire(1, 1)

    def step(it, carry):
        chunk = it * 2
        for b in range(2):
            cc = chunk + b
            drain(b)
            scatter(cc, b)

            @pl.when(cc + 2 < N_CHUNKS)
            def _():
                fire(cc + 2, b)

        return carry

    lax.fori_loop(0, N_CHUNKS // 2, step, 0)
    drain(0)
    scatter(N_CHUNKS - 1, 0)


RB = 512
NBB = B1 // (4 * RB)


def _tr_body(x_ref, o_ref):
    xt = jnp.transpose(x_ref[...])
    for q in range(4):
        for blk in range(4):
            o_ref[0, :, blk * 512 + q * 128:blk * 512 + (q + 1) * 128] = (
                xt[32 * q:32 * q + 32, blk * 128:(blk + 1) * 128]
            )


@jax.jit
def _emb_lookup(idx_flat, weight):
    mesh = plsc.VectorSubcoreMesh(
        core_axis_name="c", subcore_axis_name="s", num_cores=NC, num_subcores=NS
    )
    gather = pl.kernel(
        _gather_body,
        out_type=jax.ShapeDtypeStruct((N_BLOCKS, IDX_PER_GROUP, 4, DIM), jnp.float32),
        mesh=mesh,
        scratch_types=[
            pltpu.VMEM((GROUPS_PER_W, IDX_PER_GROUP), jnp.int32),
            pltpu.VMEM((CHUNK_ROWS, DIM), jnp.float32),
            pltpu.VMEM((CHUNK_ROWS, DIM), jnp.float32),
            pltpu.SemaphoreType.DMA,
            pltpu.SemaphoreType.DMA,
        ],
        compiler_params=pltpu.CompilerParams(use_tc_tiling_on_sc=False),
    )
    rows = gather(idx_flat, weight)

    rows_v = rows.reshape(B_TOTAL * DIM // 128, 128)

    out_t = pl.pallas_call(
        _tr_body,
        grid=(S, NBB),
        in_specs=[
            pl.BlockSpec((RB, 128), lambda si, bi: (si * NBB + bi, 0)),
        ],
        out_specs=pl.BlockSpec((1, DIM, 4 * RB), lambda si, bi: (si, 0, bi)),
        out_shape=jax.ShapeDtypeStruct((S, DIM, B1), jnp.float32),
    )(rows_v)

    return jnp.transpose(out_t, (2, 0, 1))


def kernel(input_, weight):
    idx_flat = (
        input_.astype(jnp.int32).T.reshape(B_TOTAL // IDX_PER_GROUP, IDX_PER_GROUP)
    )
    return _emb_lookup(idx_flat, weight)


# speedup vs baseline: 4.6374x; 1.0026x over previous
import jax
import jax.numpy as jnp
from jax import lax
from jax.experimental import pallas as pl
from jax.experimental.pallas import tpu as pltpu
from jax.experimental.pallas import tpu_sc as plsc

NUM_EMB = 1000000
DIM = 32
B1 = 16384
S = 50
B_TOTAL = B1 * S
NC, NS = 2, 16
NW = NC * NS
IDX_PER_GROUP = 128
GROUPS_PER_W = B_TOTAL // (NW * IDX_PER_GROUP)
G_PER_CHUNK = 8
CHUNK_ROWS = G_PER_CHUNK * IDX_PER_GROUP
N_CHUNKS = GROUPS_PER_W // G_PER_CHUNK
N_BLOCKS = B_TOTAL // 512


def _gather_body(idx_hbm, table_hbm, out_hbm, idx_v, buf0, buf1, gsem0, gsem1):
    c = lax.axis_index("c")
    s = lax.axis_index("s")
    wid = s * NC + c
    gbase = wid * GROUPS_PER_W
    bbase = wid * (GROUPS_PER_W // 4)

    pltpu.sync_copy(idx_hbm.at[pl.ds(gbase, GROUPS_PER_W)], idx_v)

    bufs = (buf0, buf1)
    gsems = (gsem0, gsem1)

    def fire(chunk, b):
        for j in range(G_PER_CHUNK):
            pltpu.async_copy(
                table_hbm.at[idx_v.at[chunk * G_PER_CHUNK + j]],
                bufs[b].at[pl.ds(j * IDX_PER_GROUP, IDX_PER_GROUP)],
                gsems[b],
            )

    def drain(b):
        pltpu.make_async_copy(
            table_hbm.at[pl.ds(0, CHUNK_ROWS)], bufs[b], gsems[b]
        ).wait()

    def scatter(chunk, b):
        for j in range(G_PER_CHUNK):
            blk = bbase + chunk * (G_PER_CHUNK // 4) + (j // 4)
            pltpu.sync_copy(
                bufs[b].at[pl.ds(j * IDX_PER_GROUP, IDX_PER_GROUP)],
                out_hbm.at[blk, :, j % 4, :],
            )

    fire(0, 0)
    fire(1, 1)

    def step(it, carry):
        chunk = it * 2
        for b in range(2):
            cc = chunk + b
            drain(b)
            scatter(cc, b)

            @pl.when(cc + 2 < N_CHUNKS)
            def _():
                fire(cc + 2, b)

        return carry

    lax.fori_loop(0, N_CHUNKS // 2, step, 0)
    drain(0)
    scatter(N_CHUNKS - 1, 0)


RB = 512
NBB = B1 // (4 * RB)


def _tr_body(x_ref, o_ref):
    xt = jnp.transpose(x_ref[...])
    for q in range(4):
        for blk in range(4):
            o_ref[0, :, blk * 512 + q * 128:blk * 512 + (q + 1) * 128] = (
                xt[32 * q:32 * q + 32, blk * 128:(blk + 1) * 128]
            )


@jax.jit
def _emb_lookup(idx_flat, weight):
    mesh = plsc.VectorSubcoreMesh(
        core_axis_name="c", subcore_axis_name="s", num_cores=NC, num_subcores=NS
    )
    gather = pl.kernel(
        _gather_body,
        out_type=jax.ShapeDtypeStruct((N_BLOCKS, IDX_PER_GROUP, 4, DIM), jnp.float32),
        mesh=mesh,
        scratch_types=[
            pltpu.VMEM((GROUPS_PER_W, IDX_PER_GROUP), jnp.int32),
            pltpu.VMEM((CHUNK_ROWS, DIM), jnp.float32),
            pltpu.VMEM((CHUNK_ROWS, DIM), jnp.float32),
            pltpu.SemaphoreType.DMA,
            pltpu.SemaphoreType.DMA,
        ],
        compiler_params=pltpu.CompilerParams(use_tc_tiling_on_sc=False),
    )
    rows = gather(idx_flat, weight)

    rows_v = rows.reshape(B_TOTAL * DIM // 128, 128)

    out_t = pl.pallas_call(
        _tr_body,
        grid=(S, NBB),
        in_specs=[
            pl.BlockSpec((RB, 128), lambda si, bi: (si * NBB + bi, 0)),
        ],
        out_specs=pl.BlockSpec((1, DIM, 4 * RB), lambda si, bi: (si, 0, bi)),
        out_shape=jax.ShapeDtypeStruct((S, DIM, B1), jnp.float32),
    )(rows_v)

    return jnp.transpose(out_t, (2, 0, 1))


def kernel(input_, weight):
    idx_flat = (
        input_.astype(jnp.int32).T.reshape(B_TOTAL // IDX_PER_GROUP, IDX_PER_GROUP)
    )
    w_lin = weight.reshape(NUM_EMB // 4, 4 * DIM).reshape(NUM_EMB, DIM)
    return _emb_lookup(idx_flat, w_lin)


# speedup vs baseline: 5.2678x; 1.1359x over previous
import jax
import jax.numpy as jnp
from jax import lax
from jax.experimental import pallas as pl
from jax.experimental.pallas import tpu as pltpu
from jax.experimental.pallas import tpu_sc as plsc

NUM_EMB = 1000000
DIM = 32
B1 = 16384
S = 50
B_TOTAL = B1 * S
NC, NS = 2, 16
NW = NC * NS
IDX_PER_GROUP = 128
GROUPS_PER_W = B_TOTAL // (NW * IDX_PER_GROUP)
G_PER_CHUNK = 8
CHUNK_ROWS = G_PER_CHUNK * IDX_PER_GROUP
N_CHUNKS = GROUPS_PER_W // G_PER_CHUNK
N_BLOCKS = B_TOTAL // 512


def _gather_body(idx_hbm, table_hbm, out_hbm, idx_v, buf0, buf1, gsem0, gsem1):
    c = lax.axis_index("c")
    s = lax.axis_index("s")
    wid = s * NC + c
    gbase = wid * GROUPS_PER_W
    bbase = wid * (GROUPS_PER_W // 4)

    pltpu.sync_copy(idx_hbm.at[pl.ds(gbase, GROUPS_PER_W)], idx_v)

    bufs = (buf0, buf1)
    gsems = (gsem0, gsem1)

    def fire(chunk, b):
        for j in range(G_PER_CHUNK):
            pltpu.async_copy(
                table_hbm.at[idx_v.at[chunk * G_PER_CHUNK + j]],
                bufs[b].at[pl.ds(j * IDX_PER_GROUP, IDX_PER_GROUP)],
                gsems[b],
            )

    def drain(b):
        pltpu.make_async_copy(
            table_hbm.at[pl.ds(0, CHUNK_ROWS)], bufs[b], gsems[b]
        ).wait()

    def scatter(chunk, b):
        for j in range(G_PER_CHUNK):
            blk = bbase + chunk * (G_PER_CHUNK // 4) + (j // 4)
            pltpu.sync_copy(
                bufs[b].at[pl.ds(j * IDX_PER_GROUP, IDX_PER_GROUP)],
                out_hbm.at[blk, :, j % 4, :],
            )

    fire(0, 0)
    fire(1, 1)

    def step(it, carry):
        chunk = it * 2
        for b in range(2):
            cc = chunk + b
            drain(b)
            scatter(cc, b)

            @pl.when(cc + 2 < N_CHUNKS)
            def _():
                fire(cc + 2, b)

        return carry

    lax.fori_loop(0, N_CHUNKS // 2, step, 0)
    drain(0)
    scatter(N_CHUNKS - 1, 0)


RB = 1024
NBB = B1 // (4 * RB)


def _tr_body(x_ref, o_ref):
    xt = jnp.transpose(x_ref[...])
    for q in range(4):
        for blk in range(RB // 128):
            o_ref[0, :, blk * 512 + q * 128:blk * 512 + (q + 1) * 128] = (
                xt[32 * q:32 * q + 32, blk * 128:(blk + 1) * 128]
            )


@jax.jit
def _emb_lookup(idx_flat, weight):
    mesh = plsc.VectorSubcoreMesh(
        core_axis_name="c", subcore_axis_name="s", num_cores=NC, num_subcores=NS
    )
    gather = pl.kernel(
        _gather_body,
        out_type=jax.ShapeDtypeStruct((N_BLOCKS, IDX_PER_GROUP, 4, DIM), jnp.float32),
        mesh=mesh,
        scratch_types=[
            pltpu.VMEM((GROUPS_PER_W, IDX_PER_GROUP), jnp.int32),
            pltpu.VMEM((CHUNK_ROWS, DIM), jnp.float32),
            pltpu.VMEM((CHUNK_ROWS, DIM), jnp.float32),
            pltpu.SemaphoreType.DMA,
            pltpu.SemaphoreType.DMA,
        ],
        compiler_params=pltpu.CompilerParams(use_tc_tiling_on_sc=False),
    )
    rows = gather(idx_flat, weight)

    rows_v = rows.reshape(B_TOTAL * DIM // 128, 128)

    out_t = pl.pallas_call(
        _tr_body,
        grid=(S, NBB),
        in_specs=[
            pl.BlockSpec((RB, 128), lambda si, bi: (si * NBB + bi, 0)),
        ],
        out_specs=pl.BlockSpec((1, DIM, 4 * RB), lambda si, bi: (si, 0, bi)),
        out_shape=jax.ShapeDtypeStruct((S, DIM, B1), jnp.float32),
    )(rows_v)

    return jnp.transpose(out_t, (2, 0, 1))


def kernel(input_, weight):
    idx_flat = (
        input_.astype(jnp.int32).T.reshape(B_TOTAL // IDX_PER_GROUP, IDX_PER_GROUP)
    )
    w_lin = weight.reshape(NUM_EMB // 4, 4 * DIM).reshape(NUM_EMB, DIM)
    return _emb_lookup(idx_flat, w_lin)
